# baseline (speedup 1.0000x reference)
import functools

import jax
import jax.numpy as jnp
from jax.experimental import pallas as pl
from jax.experimental.pallas import tpu as pltpu

BS = 128
D1, D2, D3 = 256, 512, 100000
BLK = 8192
NBLK = (D3 + BLK - 1) // BLK
NEG = -1e30
K = 5
CARRY = 128
CH = 512
NC = BLK // CH


def _leaky(x):
    return jnp.where(x >= 0, x, 0.2 * x)


def _topk_body(z_ref, w1_ref, b1_ref, w2_ref, b2_ref, g2_ref, be2_ref,
               w3_ref, b3_ref, g3_ref, be3_ref,
               idx_out_ref, h2_ref, cv_ref, ci_ref):
    j = pl.program_id(0)

    @pl.when(j == 0)
    def _prologue():
        h1 = _leaky(jnp.dot(z_ref[...], w1_ref[...],
                            preferred_element_type=jnp.float32) + b1_ref[...])
        t = jnp.dot(h1, w2_ref[...],
                    preferred_element_type=jnp.float32) + b2_ref[...]
        mean = jnp.mean(t, axis=0, keepdims=True)
        var = jnp.mean((t - mean) ** 2, axis=0, keepdims=True)
        h2 = (t - mean) * jax.lax.rsqrt(var + 0.8) * g2_ref[...] + be2_ref[...]
        h2_ref[...] = _leaky(h2)
        cv_ref[...] = jnp.full((BS, CARRY), NEG, dtype=jnp.float32)
        ci_ref[...] = jnp.zeros((BS, CARRY), dtype=jnp.int32)

    x = jnp.dot(h2_ref[...], w3_ref[...],
                preferred_element_type=jnp.float32) + b3_ref[...]
    mean3 = jnp.mean(x, axis=0, keepdims=True)
    var3 = jnp.mean(x * x, axis=0, keepdims=True) - mean3 * mean3
    scale = jax.lax.rsqrt(var3 + 0.8) * g3_ref[...]
    shift = be3_ref[...] - mean3 * scale
    xn = x * scale + shift

    base = j * BLK
    cols_row = base + jax.lax.broadcasted_iota(jnp.int32, (1, BLK), 1)
    xn = jnp.where(cols_row < D3, xn, NEG)

    cv = cv_ref[...]
    ci = ci_ref[...]
    lane_c = jax.lax.broadcasted_iota(jnp.int32, (BS, CARRY), 1)

    chunks = [xn[:, c * CH:(c + 1) * CH] for c in range(NC)]
    l_nc = jax.lax.broadcasted_iota(jnp.int32, (BS, NC), 1)
    l_ch = jax.lax.broadcasted_iota(jnp.int32, (BS, CH), 1)
    cm = jnp.full((BS, NC), NEG, dtype=jnp.float32)
    for c in range(NC):
        cm = jnp.where(l_nc == c, jnp.max(chunks[c], axis=1)[:, None], cm)
    prev_c, prev_l, bv, bi = [], [], [], []
    for t in range(K):
        m = jnp.max(cm, axis=1)
        cstar = jnp.min(jnp.where(cm == m[:, None], l_nc, NC), axis=1)
        sel = jnp.full((BS, CH), NEG, dtype=jnp.float32)
        for c in range(NC):
            sel = jnp.where((cstar == c)[:, None], chunks[c], sel)
        for p in range(t):
            sel = jnp.where((prev_c[p] == cstar)[:, None]
                            & (l_ch == prev_l[p][:, None]), NEG, sel)
        lstar = jnp.min(jnp.where(sel == m[:, None], l_ch, CH), axis=1)
        newmax = jnp.max(jnp.where(l_ch == lstar[:, None], NEG, sel), axis=1)
        cm = jnp.where(l_nc == cstar[:, None], newmax[:, None], cm)
        bv.append(m)
        bi.append(base + cstar * CH + lstar)
        prev_c.append(cstar)
        prev_l.append(lstar)

    cand_v = cv
    cand_i = ci
    for p in range(K):
        cand_v = jnp.where(lane_c == K + p, bv[p][:, None], cand_v)
        cand_i = jnp.where(lane_c == K + p, bi[p][:, None], cand_i)
    newv = jnp.full((BS, CARRY), NEG, dtype=jnp.float32)
    newi = jnp.zeros((BS, CARRY), dtype=jnp.int32)
    for t in range(K):
        m = jnp.max(cand_v, axis=1)
        am = jnp.min(jnp.where(cand_v == m[:, None], lane_c, CARRY), axis=1)
        gi = jnp.sum(jnp.where(lane_c == am[:, None], cand_i, 0), axis=1)
        newv = jnp.where(lane_c == t, m[:, None], newv)
        newi = jnp.where(lane_c == t, gi[:, None], newi)
        cand_v = jnp.where(lane_c == am[:, None], NEG, cand_v)

    cv_ref[...] = newv
    ci_ref[...] = newi
    idx_out_ref[...] = newi


def _scatter_body(oh_in_ref, scat_smem_ref, scat_vec_ref,
                  oh_out_ref, strips_ref, sem):
    del oh_in_ref
    p = scat_vec_ref[...]
    lane = jax.lax.broadcasted_iota(jnp.int32, (BS * K, 128), 1)
    strips_ref[...] = jnp.where(lane == p % 128, 1.0, 0.0).astype(jnp.float32)

    def _start(i, _):
        pi = scat_smem_ref[i]
        r = i // K
        t = i - r * K
        cb = (pi // 128) * 128
        pltpu.make_async_copy(
            strips_ref.at[i], oh_out_ref.at[r, t, pl.ds(cb, 128)], sem
        ).start()
        return 0

    jax.lax.fori_loop(0, BS * K, _start, 0)

    def _drain(i, _):
        pltpu.make_async_copy(
            strips_ref.at[0], oh_out_ref.at[0, 0, pl.ds(0, 128)], sem
        ).wait()
        return 0

    jax.lax.fori_loop(0, BS * K, _drain, 0)


@functools.partial(jax.jit, static_argnums=(0,))
def _run(bs_static, z, W1, b1, W2, b2, gamma2, beta2, W3, b3, gamma3, beta3):
    full = lambda shape: pl.BlockSpec(shape, lambda j: (0, 0))
    colblk = lambda r: pl.BlockSpec((r, BLK), lambda j: (0, j))

    idx_pad = pl.pallas_call(
        _topk_body,
        grid=(NBLK,),
        in_specs=[
            full((BS, BS)),
            full((BS, D1)),
            full((1, D1)),
            full((D1, D2)),
            full((1, D2)),
            full((1, D2)),
            full((1, D2)),
            colblk(D2),
            colblk(1),
            colblk(1),
            colblk(1),
        ],
        out_specs=pl.BlockSpec((BS, CARRY), lambda j: (0, 0)),
        out_shape=jax.ShapeDtypeStruct((BS, CARRY), jnp.int32),
        scratch_shapes=[
            pltpu.VMEM((BS, D2), jnp.float32),
            pltpu.VMEM((BS, CARRY), jnp.float32),
            pltpu.VMEM((BS, CARRY), jnp.int32),
        ],
    )(z, W1, b1.reshape(1, D1), W2, b2.reshape(1, D2),
      gamma2.reshape(1, D2), beta2.reshape(1, D2),
      W3, b3.reshape(1, D3), gamma3.reshape(1, D3), beta3.reshape(1, D3))

    idx = idx_pad[:, :K]
    oh = (jnp.arange(D3, dtype=jnp.int32)[None, None, :]
          == idx[:, :, None]).astype(jnp.float32)
    return oh, idx


def kernel(bs, z, W1, b1, W2, b2, gamma2, beta2, W3, b3, gamma3, beta3):
    return _run(z.shape[0], z, W1, b1, W2, b2, gamma2, beta2,
                W3, b3, gamma3, beta3)


# speedup vs baseline: 1.0282x; 1.0282x over previous
import functools

import jax
import jax.numpy as jnp
from jax.experimental import pallas as pl
from jax.experimental.pallas import tpu as pltpu

BS = 128
D1, D2, D3 = 256, 512, 100000
BLK = 12288
NBLK = (D3 + BLK - 1) // BLK
NEG = -1e30
K = 5
CARRY = 128
CH = 512
NC = BLK // CH


def _leaky(x):
    return jnp.where(x >= 0, x, 0.2 * x)


def _topk_body(z_ref, w1_ref, b1_ref, w2_ref, b2_ref, g2_ref, be2_ref,
               w3_ref, b3_ref, g3_ref, be3_ref,
               idx_out_ref, h2_ref, cv_ref, ci_ref):
    j = pl.program_id(0)

    @pl.when(j == 0)
    def _prologue():
        h1 = _leaky(jnp.dot(z_ref[...], w1_ref[...],
                            preferred_element_type=jnp.float32) + b1_ref[...])
        t = jnp.dot(h1, w2_ref[...],
                    preferred_element_type=jnp.float32) + b2_ref[...]
        mean = jnp.mean(t, axis=0, keepdims=True)
        var = jnp.mean((t - mean) ** 2, axis=0, keepdims=True)
        h2 = (t - mean) * jax.lax.rsqrt(var + 0.8) * g2_ref[...] + be2_ref[...]
        h2_ref[...] = _leaky(h2)
        cv_ref[...] = jnp.full((BS, CARRY), NEG, dtype=jnp.float32)
        ci_ref[...] = jnp.zeros((BS, CARRY), dtype=jnp.int32)

    x = jnp.dot(h2_ref[...], w3_ref[...],
                preferred_element_type=jnp.float32) + b3_ref[...]
    mean3 = jnp.mean(x, axis=0, keepdims=True)
    var3 = jnp.mean(x * x, axis=0, keepdims=True) - mean3 * mean3
    scale = jax.lax.rsqrt(var3 + 0.8) * g3_ref[...]
    shift = be3_ref[...] - mean3 * scale
    xn = x * scale + shift

    base = j * BLK
    cols_row = base + jax.lax.broadcasted_iota(jnp.int32, (1, BLK), 1)
    xn = jnp.where(cols_row < D3, xn, NEG)

    cv = cv_ref[...]
    ci = ci_ref[...]
    lane_c = jax.lax.broadcasted_iota(jnp.int32, (BS, CARRY), 1)

    chunks = [xn[:, c * CH:(c + 1) * CH] for c in range(NC)]
    l_nc = jax.lax.broadcasted_iota(jnp.int32, (BS, NC), 1)
    l_ch = jax.lax.broadcasted_iota(jnp.int32, (BS, CH), 1)
    cm = jnp.full((BS, NC), NEG, dtype=jnp.float32)
    for c in range(NC):
        cm = jnp.where(l_nc == c, jnp.max(chunks[c], axis=1)[:, None], cm)
    prev_c, prev_l, bv, bi = [], [], [], []
    for t in range(K):
        m = jnp.max(cm, axis=1)
        cstar = jnp.min(jnp.where(cm == m[:, None], l_nc, NC), axis=1)
        sel = jnp.full((BS, CH), NEG, dtype=jnp.float32)
        for c in range(NC):
            sel = jnp.where((cstar == c)[:, None], chunks[c], sel)
        for p in range(t):
            sel = jnp.where((prev_c[p] == cstar)[:, None]
                            & (l_ch == prev_l[p][:, None]), NEG, sel)
        lstar = jnp.min(jnp.where(sel == m[:, None], l_ch, CH), axis=1)
        newmax = jnp.max(jnp.where(l_ch == lstar[:, None], NEG, sel), axis=1)
        cm = jnp.where(l_nc == cstar[:, None], newmax[:, None], cm)
        bv.append(m)
        bi.append(base + cstar * CH + lstar)
        prev_c.append(cstar)
        prev_l.append(lstar)

    cand_v = cv
    cand_i = ci
    for p in range(K):
        cand_v = jnp.where(lane_c == K + p, bv[p][:, None], cand_v)
        cand_i = jnp.where(lane_c == K + p, bi[p][:, None], cand_i)
    newv = jnp.full((BS, CARRY), NEG, dtype=jnp.float32)
    newi = jnp.zeros((BS, CARRY), dtype=jnp.int32)
    for t in range(K):
        m = jnp.max(cand_v, axis=1)
        am = jnp.min(jnp.where(cand_v == m[:, None], lane_c, CARRY), axis=1)
        gi = jnp.sum(jnp.where(lane_c == am[:, None], cand_i, 0), axis=1)
        newv = jnp.where(lane_c == t, m[:, None], newv)
        newi = jnp.where(lane_c == t, gi[:, None], newi)
        cand_v = jnp.where(lane_c == am[:, None], NEG, cand_v)

    cv_ref[...] = newv
    ci_ref[...] = newi
    idx_out_ref[...] = newi


def _scatter_body(oh_in_ref, scat_smem_ref, scat_vec_ref,
                  oh_out_ref, strips_ref, sem):
    del oh_in_ref
    p = scat_vec_ref[...]
    lane = jax.lax.broadcasted_iota(jnp.int32, (BS * K, 128), 1)
    strips_ref[...] = jnp.where(lane == p % 128, 1.0, 0.0).astype(jnp.float32)

    def _start(i, _):
        pi = scat_smem_ref[i]
        r = i // K
        t = i - r * K
        cb = (pi // 128) * 128
        pltpu.make_async_copy(
            strips_ref.at[i], oh_out_ref.at[r, t, pl.ds(cb, 128)], sem
        ).start()
        return 0

    jax.lax.fori_loop(0, BS * K, _start, 0)

    def _drain(i, _):
        pltpu.make_async_copy(
            strips_ref.at[0], oh_out_ref.at[0, 0, pl.ds(0, 128)], sem
        ).wait()
        return 0

    jax.lax.fori_loop(0, BS * K, _drain, 0)


@functools.partial(jax.jit, static_argnums=(0,))
def _run(bs_static, z, W1, b1, W2, b2, gamma2, beta2, W3, b3, gamma3, beta3):
    full = lambda shape: pl.BlockSpec(shape, lambda j: (0, 0))
    colblk = lambda r: pl.BlockSpec((r, BLK), lambda j: (0, j))

    idx_pad = pl.pallas_call(
        _topk_body,
        grid=(NBLK,),
        in_specs=[
            full((BS, BS)),
            full((BS, D1)),
            full((1, D1)),
            full((D1, D2)),
            full((1, D2)),
            full((1, D2)),
            full((1, D2)),
            colblk(D2),
            colblk(1),
            colblk(1),
            colblk(1),
        ],
        out_specs=pl.BlockSpec((BS, CARRY), lambda j: (0, 0)),
        out_shape=jax.ShapeDtypeStruct((BS, CARRY), jnp.int32),
        scratch_shapes=[
            pltpu.VMEM((BS, D2), jnp.float32),
            pltpu.VMEM((BS, CARRY), jnp.float32),
            pltpu.VMEM((BS, CARRY), jnp.int32),
        ],
    )(z, W1, b1.reshape(1, D1), W2, b2.reshape(1, D2),
      gamma2.reshape(1, D2), beta2.reshape(1, D2),
      W3, b3.reshape(1, D3), gamma3.reshape(1, D3), beta3.reshape(1, D3))

    idx = idx_pad[:, :K]
    oh = (jnp.arange(D3, dtype=jnp.int32)[None, None, :]
          == idx[:, :, None]).astype(jnp.float32)
    return oh, idx


def kernel(bs, z, W1, b1, W2, b2, gamma2, beta2, W3, b3, gamma3, beta3):
    return _run(z.shape[0], z, W1, b1, W2, b2, gamma2, beta2,
                W3, b3, gamma3, beta3)
